# baseline (speedup 1.0000x reference)
import functools

import jax
import jax.numpy as jnp
from jax import lax
from jax.experimental import pallas as pl
from jax.experimental.pallas import tpu as pltpu
from jax.experimental.pallas import tpu_sc as plsc

_B = 16384
_N = 11
_D = 32
_NC = 2
_NS = 16
_NW = _NC * _NS
_PER_W = _B // _NW
_CH = 128
_NCH = _PER_W // _CH
_LANES = 16


def _recsys_call(sess_flat, prods2d, uemb, pemb, *, interpret=False):
    mesh = plsc.VectorSubcoreMesh(
        core_axis_name="c", subcore_axis_name="s",
        num_cores=_NC, num_subcores=_NS)

    @functools.partial(
        pl.kernel,
        out_type=jax.ShapeDtypeStruct((_B, _N), jnp.float32),
        mesh=mesh,
        compiler_params=pltpu.CompilerParams(
            use_tc_tiling_on_sc=False, needs_layout_passes=False),
        scratch_types=[
            pltpu.VMEM((_CH,), jnp.int32),
            pltpu.VMEM((_N * _CH,), jnp.int32),
            pltpu.VMEM((_CH, _D), jnp.float32),
            pltpu.VMEM((_CH * _N, _D), jnp.float32),
            pltpu.VMEM((_CH, _N), jnp.float32),
            pltpu.SemaphoreType.DMA,
            pltpu.SemaphoreType.DMA,
        ],
        interpret=interpret,
    )
    def body(sess_hbm, prods_hbm, uemb_hbm, pemb_hbm, out_hbm,
             sidx, pidx, urows, prows, outv, usem, psem):
        wid = lax.axis_index("c") * _NS + lax.axis_index("s")

        def chunk_body(c, carry):
            gbase = wid * _PER_W + c * _CH
            pltpu.sync_copy(sess_hbm.at[pl.ds(gbase, _CH)], sidx)
            pltpu.sync_copy(
                prods_hbm.at[pl.ds(gbase * _N, _N * _CH)], pidx)

            ucp = pltpu.make_async_copy(uemb_hbm.at[sidx], urows, usem)
            ucp.start()
            pcps = []
            for j in range(_N):
                cp = pltpu.make_async_copy(
                    pemb_hbm.at[pidx.at[pl.ds(j * _CH, _CH)]],
                    prows.at[pl.ds(j * _CH, _CH)], psem)
                cp.start()
                pcps.append(cp)
            ucp.wait()
            for cp in pcps:
                cp.wait()

            def group_body(g, carry2):
                bvec = g * _LANES + lax.iota(jnp.int32, _LANES)
                us = [
                    plsc.load_gather(
                        urows, [bvec, jnp.full((_LANES,), d, jnp.int32)])
                    for d in range(_D)
                ]
                for n in range(_N):
                    qvec = bvec * _N + n
                    acc = jnp.zeros((_LANES,), jnp.float32)
                    for d in range(_D):
                        pv = plsc.load_gather(
                            prows, [qvec, jnp.full((_LANES,), d, jnp.int32)])
                        acc = acc + us[d] * pv
                    plsc.store_scatter(
                        outv, [bvec, jnp.full((_LANES,), n, jnp.int32)], acc)
                return carry2

            lax.fori_loop(0, _CH // _LANES, group_body, 0)
            pltpu.sync_copy(outv, out_hbm.at[pl.ds(gbase, _CH)])
            return carry

        lax.fori_loop(0, _NCH, chunk_body, 0)

    return body(sess_flat, prods2d, uemb, pemb)


def kernel(session, products, user_embedding, product_embedding):
    sess_flat = session.reshape(-1)
    prods_flat = products.reshape(-1)
    return _recsys_call(sess_flat, prods_flat, user_embedding,
                        product_embedding)


# speedup vs baseline: 1.3181x; 1.3181x over previous
import functools

import jax
import jax.numpy as jnp
from jax import lax
from jax.experimental import pallas as pl
from jax.experimental.pallas import tpu as pltpu
from jax.experimental.pallas import tpu_sc as plsc

_B = 16384
_N = 11
_D = 32
_NC = 2
_NS = 16
_NW = _NC * _NS
_PER_W = _B // _NW
_CH = 128
_NCH = _PER_W // _CH
_LANES = 16


def _user_gather_call(uemb, sess_flat):
    mesh = plsc.VectorSubcoreMesh(
        core_axis_name="c", subcore_axis_name="s",
        num_cores=_NC, num_subcores=_NS)

    @functools.partial(
        pl.kernel,
        out_type=jax.ShapeDtypeStruct((_B, _D), jnp.float32),
        mesh=mesh,
        compiler_params=pltpu.CompilerParams(needs_layout_passes=False),
        scratch_types=[
            pltpu.VMEM((_PER_W,), jnp.int32),
            pltpu.VMEM((_LANES, 8, _D), jnp.float32),
            pltpu.VMEM((_LANES, _D), jnp.float32),
            pltpu.SemaphoreType.DMA,
        ],
    )
    def body(uemb_hbm, sess_hbm, out_hbm, sidx, tbuf, ubuf, sem):
        wid = lax.axis_index("c") * _NS + lax.axis_index("s")
        base = wid * _PER_W
        iota16 = lax.iota(jnp.int32, _LANES)
        pltpu.sync_copy(sess_hbm.at[pl.ds(base, _PER_W)], sidx)

        def group_body(g, carry):
            svec = sidx[pl.ds(g * _LANES, _LANES)]
            sub = svec % 8
            cps = []
            for l in range(_LANES):
                r = jnp.sum(jnp.where(iota16 == l, svec, 0))
                r0 = pl.multiple_of((r // 8) * 8, 8)
                cp = pltpu.make_async_copy(
                    uemb_hbm.at[pl.ds(r0, 8), :], tbuf.at[l], sem)
                cp.start()
                cps.append(cp)
            for cp in cps:
                cp.wait()
            for l in range(_LANES):
                s = jnp.sum(jnp.where(iota16 == l, sub, 0))
                ubuf[l, pl.ds(0, _LANES)] = tbuf[l, s, pl.ds(0, _LANES)]
                ubuf[l, pl.ds(_LANES, _LANES)] = (
                    tbuf[l, s, pl.ds(_LANES, _LANES)])
            pltpu.sync_copy(
                ubuf, out_hbm.at[pl.ds(base + g * _LANES, _LANES)])
            return carry

        lax.fori_loop(0, _PER_W // _LANES, group_body, 0)

    return body(uemb, sess_flat)


def _score_call(urows, prods_flat, pemb):
    mesh = plsc.VectorSubcoreMesh(
        core_axis_name="c", subcore_axis_name="s",
        num_cores=_NC, num_subcores=_NS)

    @functools.partial(
        pl.kernel,
        out_type=jax.ShapeDtypeStruct((_B, _N), jnp.float32),
        mesh=mesh,
        compiler_params=pltpu.CompilerParams(
            use_tc_tiling_on_sc=False, needs_layout_passes=False),
        scratch_types=[
            pltpu.VMEM((_N * _CH,), jnp.int32),
            pltpu.VMEM((_CH, _D), jnp.float32),
            pltpu.VMEM((_CH * _N, _D), jnp.float32),
            pltpu.VMEM((_CH, _N), jnp.float32),
            pltpu.SemaphoreType.DMA,
            pltpu.SemaphoreType.DMA,
        ],
    )
    def body(urows_hbm, prods_hbm, pemb_hbm, out_hbm,
             pidx, ubuf, prows, outv, usem, psem):
        wid = lax.axis_index("c") * _NS + lax.axis_index("s")

        def chunk_body(c, carry):
            gbase = wid * _PER_W + c * _CH
            pltpu.sync_copy(
                prods_hbm.at[pl.ds(gbase * _N, _N * _CH)], pidx)
            ucp = pltpu.make_async_copy(
                urows_hbm.at[pl.ds(gbase, _CH)], ubuf, usem)
            ucp.start()
            pcps = []
            for j in range(_N):
                cp = pltpu.make_async_copy(
                    pemb_hbm.at[pidx.at[pl.ds(j * _CH, _CH)]],
                    prows.at[pl.ds(j * _CH, _CH)], psem)
                cp.start()
                pcps.append(cp)
            ucp.wait()
            for cp in pcps:
                cp.wait()

            def group_body(g, carry2):
                bvec = g * _LANES + lax.iota(jnp.int32, _LANES)
                us = [
                    plsc.load_gather(
                        ubuf, [bvec, jnp.full((_LANES,), d, jnp.int32)])
                    for d in range(_D)
                ]
                for n in range(_N):
                    qvec = bvec * _N + n
                    acc = jnp.zeros((_LANES,), jnp.float32)
                    for d in range(_D):
                        pv = plsc.load_gather(
                            prows, [qvec, jnp.full((_LANES,), d, jnp.int32)])
                        acc = acc + us[d] * pv
                    plsc.store_scatter(
                        outv, [bvec, jnp.full((_LANES,), n, jnp.int32)], acc)
                return carry2

            lax.fori_loop(0, _CH // _LANES, group_body, 0)
            pltpu.sync_copy(outv, out_hbm.at[pl.ds(gbase, _CH)])
            return carry

        lax.fori_loop(0, _NCH, chunk_body, 0)

    return body(urows, prods_flat, pemb)


def kernel(session, products, user_embedding, product_embedding):
    sess_flat = session.reshape(-1)
    prods_flat = products.reshape(-1)
    urows = _user_gather_call(user_embedding, sess_flat)
    return _score_call(urows, prods_flat, product_embedding)
